# baseline (speedup 1.0000x reference)
import dataclasses
import functools

import jax
import jax.numpy as jnp
from jax import lax
from jax.experimental import pallas as pl
from jax.experimental.pallas import tpu as pltpu
from jax.experimental.pallas import tpu_sc as plsc

N_NODES = 10000
N_EDGES = 320000
D_IN = 128
D_H = 128
NC = 2
NS = 16
NW = NC * NS
LANES = 16
C = 64
NPT = 160
NE_PAD = NW * C * NPT
S_ROWS = 10048
ZB = S_ROWS // C
OUT_BLOCKS = N_NODES // C
TAIL_ROWS = N_NODES - OUT_BLOCKS * C
H_ROWS = 80
HIGH = jax.lax.Precision.HIGHEST

_sc_mesh = plsc.VectorSubcoreMesh(
    core_axis_name="c", subcore_axis_name="s", num_cores=NC, num_subcores=NS)

_sc_params = pltpu.CompilerParams()
if "needs_layout_passes" in pltpu.CompilerParams.__dataclass_fields__:
    _sc_params = dataclasses.replace(_sc_params, needs_layout_passes=False)


@functools.partial(
    pl.kernel,
    out_type=(jax.ShapeDtypeStruct((NC, N_NODES, D_H), jnp.float32),
              jax.ShapeDtypeStruct((NC, H_ROWS, D_H), jnp.float32)),
    mesh=_sc_mesh,
    compiler_params=_sc_params,
    scratch_types=[
        pltpu.VMEM((C,), jnp.int32), pltpu.VMEM((C,), jnp.int32),
        pltpu.VMEM((C,), jnp.int32), pltpu.VMEM((C,), jnp.int32),
        pltpu.VMEM((C,), jnp.int32), pltpu.VMEM((C,), jnp.int32),
        pltpu.VMEM((C,), jnp.int32), pltpu.VMEM((C,), jnp.int32),
        pltpu.VMEM((C, D_H), jnp.float32), pltpu.VMEM((C, D_H), jnp.float32),
        pltpu.VMEM((C, D_H), jnp.float32), pltpu.VMEM((C, D_H), jnp.float32),
        pltpu.VMEM((H_ROWS, D_H), jnp.float32),
        pltpu.VMEM((H_ROWS,), jnp.int32),
        pltpu.SemaphoreType.DMA, pltpu.SemaphoreType.DMA,
        pltpu.SemaphoreType.DMA, pltpu.SemaphoreType.DMA,
        pltpu.SemaphoreType.DMA, pltpu.SemaphoreType.DMA,
        pltpu.SemaphoreType.DMA, pltpu.SemaphoreType.DMA,
        pltpu.VMEM_SHARED((S_ROWS, D_H), jnp.float32),
        pltpu.VMEM_SHARED((H_ROWS, D_H), jnp.float32),
    ],
)
def _sc_segment_kernel(p_hbm, e_hbm, col_hbm, row_hbm, s_out, cnt_out,
                       cb0, cb1, cb2, cb3, rb0, rb1, rb2, rb3,
                       pbA, pbB, ebA, ebB, histbuf, iotabuf,
                       dsA, dsB, cs0, cs1, cs2, cs3, ssA, ssB,
                       s_shared, cnt_shared):
    cid = lax.axis_index("c")
    sid = lax.axis_index("s")
    wid = sid * NC + cid
    zeros16 = jnp.zeros((LANES,), jnp.float32)
    ones16 = jnp.ones((LANES,), jnp.float32)
    lane16 = lax.iota(jnp.int32, 16)

    @pl.loop(0, C)
    def _(r):
        for k in range(D_H // LANES):
            pbA[r, pl.ds(k * LANES, LANES)] = zeros16

    @pl.loop(0, H_ROWS)
    def _(r):
        for k in range(D_H // LANES):
            histbuf[r, pl.ds(k * LANES, LANES)] = zeros16

    for k in range(H_ROWS // LANES):
        iotabuf[pl.ds(k * LANES, LANES)] = lane16 + (k * LANES)

    @pl.loop(sid, ZB, step=NS)
    def _(b):
        pltpu.sync_copy(pbA, s_shared.at[pl.ds(b * C, C)])

    @pl.when(sid == 1)
    def _():
        pltpu.sync_copy(pbA.at[pl.ds(0, H_ROWS)], cnt_shared)

    plsc.subcore_barrier()

    cbs = (cb0, cb1, cb2, cb3)
    rbs = (rb0, rb1, rb2, rb3)
    css = (cs0, cs1, cs2, cs3)

    def fetch_colrow(chunk, q):
        base = chunk * C
        pltpu.async_copy(col_hbm.at[pl.ds(base, C)], cbs[q], css[q])
        pltpu.async_copy(row_hbm.at[pl.ds(base, C)], rbs[q], css[q])

    def do_chunk(j, t):
        chunk = wid + NW * j
        pb, eb, ds, ss = ((pbA, ebA, dsA, ssA) if t % 2 == 0
                          else (pbB, ebB, dsB, ssB))
        npb, neb, nds, nss = ((pbA, ebA, dsA, ssA) if t % 2 == 1
                             else (pbB, ebB, dsB, ssB))
        q, nq, q2 = t % 4, (t + 1) % 4, (t + 2) % 4

        pltpu.make_async_copy(p_hbm.at[cbs[q]], pb, ds).wait()
        pltpu.make_async_copy(e_hbm.at[pl.ds(0, C)], eb, ds).wait()

        @pl.when(j >= 1)
        def _():
            pltpu.make_async_copy(npb, s_shared.at[rbs[q]], nss).wait()

        @pl.when(j + 1 < NPT)
        def _():
            pltpu.make_async_copy(col_hbm.at[pl.ds(0, C)], cbs[nq],
                                  css[nq]).wait()
            pltpu.make_async_copy(row_hbm.at[pl.ds(0, C)], rbs[nq],
                                  css[nq]).wait()
            pltpu.async_copy(p_hbm.at[cbs[nq]], npb, nds)
            pltpu.async_copy(e_hbm.at[pl.ds((chunk + NW) * C, C)], neb, nds)

        @pl.when(j + 2 < NPT)
        def _():
            fetch_colrow(chunk + 2 * NW, q2)

        @plsc.parallel_loop(0, C, unroll=2)
        def _(r):
            for k in range(D_H // LANES):
                sl = pl.ds(k * LANES, LANES)
                pb[r, sl] = jnp.maximum(pb[r, sl] + eb[r, sl], 0.0)

        for k in range(C // LANES):
            rv = rbs[q][pl.ds(k * LANES, LANES)]
            plsc.addupdate_scatter(
                histbuf, [lax.shift_right_logical(rv, 7),
                          lax.bitwise_and(rv, 127)], ones16)

        pltpu.async_copy(pb, s_shared.at[rbs[q]], ss, add=True)

    pltpu.sync_copy(col_hbm.at[pl.ds(wid * C, C)], cb0)
    pltpu.sync_copy(row_hbm.at[pl.ds(wid * C, C)], rb0)
    pltpu.async_copy(p_hbm.at[cb0], pbA, dsA)
    pltpu.async_copy(e_hbm.at[pl.ds(wid * C, C)], ebA, dsA)
    fetch_colrow(wid + NW, 1)

    @pl.loop(0, NPT, step=4)
    def _(jj):
        do_chunk(jj, 0)
        do_chunk(jj + 1, 1)
        do_chunk(jj + 2, 2)
        do_chunk(jj + 3, 3)

    pltpu.make_async_copy(pbB, s_shared.at[rb3], ssB).wait()

    pltpu.sync_copy(histbuf, cnt_shared.at[iotabuf], add=True)

    plsc.subcore_barrier()

    @pl.loop(sid, OUT_BLOCKS, step=NS)
    def _(b):
        pltpu.sync_copy(s_shared.at[pl.ds(b * C, C)],
                        s_out.at[cid].at[pl.ds(b * C, C)])

    @pl.when(sid == 0)
    def _():
        pltpu.sync_copy(s_shared.at[pl.ds(OUT_BLOCKS * C, TAIL_ROWS)],
                        s_out.at[cid].at[pl.ds(OUT_BLOCKS * C, TAIL_ROWS)])

    @pl.when(sid == 1)
    def _():
        pltpu.sync_copy(cnt_shared, cnt_out.at[cid])


def _node_proj_body(nf_ref, w_ref, b_ref, out_ref):
    out_ref[...] = lax.dot_general(
        nf_ref[...], w_ref[...], (((1,), (0,)), ((), ())),
        preferred_element_type=jnp.float32, precision=HIGH) + b_ref[...]


def _edge_proj_body(ea_ref, w_ref, out_ref):
    out_ref[...] = lax.dot_general(
        ea_ref[...], w_ref[...], (((1,), (0,)), ((), ())),
        preferred_element_type=jnp.float32,
        precision=jax.lax.Precision.DEFAULT)


def _final_body(p0_ref, p1_ref, c0_ref, c1_ref, nf_ref, w2_ref, b2_ref,
                w3n_ref, w3m_ref, b3_ref, w4_ref, b4_ref, out_ref):
    s = p0_ref[...] + p1_ref[...]
    cnt = c0_ref[...] + c1_ref[...]
    sum_t = lax.dot_general(s, w2_ref[...], (((1,), (0,)), ((), ())),
                            preferred_element_type=jnp.float32,
                            precision=HIGH) + cnt * b2_ref[...]
    agg = sum_t / jnp.maximum(cnt, 1.0)
    u = lax.dot_general(nf_ref[...], w3n_ref[...], (((1,), (0,)), ((), ())),
                        preferred_element_type=jnp.float32, precision=HIGH)
    u = u + lax.dot_general(agg, w3m_ref[...], (((1,), (0,)), ((), ())),
                            preferred_element_type=jnp.float32,
                            precision=HIGH) + b3_ref[...]
    u = jnp.maximum(u, 0.0)
    out_ref[...] = lax.dot_general(
        u, w4_ref[...], (((1,), (0,)), ((), ())),
        preferred_element_type=jnp.float32, precision=HIGH) + b4_ref[...]


def kernel(node_feat, edge_index, edge_attr, W1, b1, W2, b2, W3, b3, W4, b4):
    pad = NE_PAD - N_EDGES
    col = jnp.concatenate([edge_index[1], jnp.zeros((pad,), jnp.int32)])
    row = jnp.concatenate([edge_index[0],
                           jnp.full((pad,), N_NODES, jnp.int32)])
    ea = jnp.concatenate([edge_attr, jnp.zeros((pad, 16), jnp.float32)])
    w1n, w1e = W1[:D_IN], W1[D_IN:]
    w3n, w3m = W3[:D_IN], W3[D_IN:]

    p = pl.pallas_call(
        _node_proj_body,
        out_shape=jax.ShapeDtypeStruct((N_NODES, D_H), jnp.float32),
    )(node_feat, w1n, b1.reshape(1, D_H))

    eb = 4096
    e = pl.pallas_call(
        _edge_proj_body,
        grid=(NE_PAD // eb,),
        in_specs=[pl.BlockSpec((eb, 16), lambda i: (i, 0)),
                  pl.BlockSpec((16, D_H), lambda i: (0, 0))],
        out_specs=pl.BlockSpec((eb, D_H), lambda i: (i, 0)),
        out_shape=jax.ShapeDtypeStruct((NE_PAD, D_H), jnp.float32),
    )(ea, w1e)

    partials, counts = _sc_segment_kernel(p, e, col, row)
    cnt0 = counts[0].reshape(H_ROWS * D_H, 1)[:N_NODES]
    cnt1 = counts[1].reshape(H_ROWS * D_H, 1)[:N_NODES]

    nb = 1000
    out = pl.pallas_call(
        _final_body,
        grid=(N_NODES // nb,),
        in_specs=[pl.BlockSpec((nb, D_H), lambda i: (i, 0)),
                  pl.BlockSpec((nb, D_H), lambda i: (i, 0)),
                  pl.BlockSpec((nb, 1), lambda i: (i, 0)),
                  pl.BlockSpec((nb, 1), lambda i: (i, 0)),
                  pl.BlockSpec((nb, D_IN), lambda i: (i, 0)),
                  pl.BlockSpec((D_H, D_H), lambda i: (0, 0)),
                  pl.BlockSpec((1, D_H), lambda i: (0, 0)),
                  pl.BlockSpec((D_IN, D_H), lambda i: (0, 0)),
                  pl.BlockSpec((D_H, D_H), lambda i: (0, 0)),
                  pl.BlockSpec((1, D_H), lambda i: (0, 0)),
                  pl.BlockSpec((D_H, D_H), lambda i: (0, 0)),
                  pl.BlockSpec((1, D_H), lambda i: (0, 0))],
        out_specs=pl.BlockSpec((nb, D_H), lambda i: (i, 0)),
        out_shape=jax.ShapeDtypeStruct((N_NODES, D_H), jnp.float32),
    )(partials[0], partials[1], cnt0, cnt1, node_feat, W2, b2.reshape(1, D_H),
      w3n, w3m, b3.reshape(1, D_H), W4, b4.reshape(1, D_H))

    return out


# speedup vs baseline: 1.4090x; 1.4090x over previous
import dataclasses
import functools

import jax
import jax.numpy as jnp
from jax import lax
from jax.experimental import pallas as pl
from jax.experimental.pallas import tpu as pltpu
from jax.experimental.pallas import tpu_sc as plsc

N_NODES = 10000
N_EDGES = 320000
D_IN = 128
D_H = 128
NC = 2
NS = 16
NW = NC * NS
LANES = 16
C = 40
NPT = N_EDGES // (NW * C)
ZB = N_NODES // C
H_ROWS = 80
HIGH = jax.lax.Precision.HIGHEST

_sc_mesh = plsc.VectorSubcoreMesh(
    core_axis_name="c", subcore_axis_name="s", num_cores=NC, num_subcores=NS)

_sc_params = pltpu.CompilerParams()
_flds = pltpu.CompilerParams.__dataclass_fields__
if "needs_layout_passes" in _flds:
    _sc_params = dataclasses.replace(_sc_params, needs_layout_passes=False)
if "use_tc_tiling_on_sc" in _flds:
    _sc_params = dataclasses.replace(_sc_params, use_tc_tiling_on_sc=True)


@functools.partial(
    pl.kernel,
    out_type=(jax.ShapeDtypeStruct((NC, N_NODES, D_H), jnp.float32),
              jax.ShapeDtypeStruct((NC, H_ROWS, D_H), jnp.float32)),
    mesh=_sc_mesh,
    compiler_params=_sc_params,
    scratch_types=[
        pltpu.VMEM((C,), jnp.int32), pltpu.VMEM((C,), jnp.int32),
        pltpu.VMEM((C,), jnp.int32), pltpu.VMEM((C,), jnp.int32),
        pltpu.VMEM((C,), jnp.int32), pltpu.VMEM((C,), jnp.int32),
        pltpu.VMEM((C,), jnp.int32), pltpu.VMEM((C,), jnp.int32),
        pltpu.VMEM((C, D_H), jnp.float32), pltpu.VMEM((C, D_H), jnp.float32),
        pltpu.VMEM((C, D_H), jnp.float32), pltpu.VMEM((C, D_H), jnp.float32),
        pltpu.VMEM((H_ROWS, D_H), jnp.float32),
        pltpu.VMEM((H_ROWS,), jnp.int32),
        pltpu.SemaphoreType.DMA, pltpu.SemaphoreType.DMA,
        pltpu.SemaphoreType.DMA, pltpu.SemaphoreType.DMA,
        pltpu.SemaphoreType.DMA, pltpu.SemaphoreType.DMA,
        pltpu.SemaphoreType.DMA, pltpu.SemaphoreType.DMA,
        pltpu.VMEM_SHARED((N_NODES, D_H), jnp.float32),
        pltpu.VMEM_SHARED((H_ROWS, D_H), jnp.float32),
    ],
)
def _sc_segment_kernel(p_hbm, e_hbm, col_hbm, row_hbm, s_out, cnt_out,
                       cb0, cb1, cb2, cb3, rb0, rb1, rb2, rb3,
                       pbA, pbB, ebA, ebB, histbuf, iotabuf,
                       dsA, dsB, cs0, cs1, cs2, cs3, ssA, ssB,
                       s_shared, cnt_shared):
    cid = lax.axis_index("c")
    sid = lax.axis_index("s")
    wid = sid * NC + cid
    zeros16 = jnp.zeros((LANES,), jnp.float32)
    ones16 = jnp.ones((LANES,), jnp.float32)
    lane16 = lax.iota(jnp.int32, 16)

    @pl.loop(0, C)
    def _(r):
        for k in range(D_H // LANES):
            pbA[r, pl.ds(k * LANES, LANES)] = zeros16

    @pl.loop(0, H_ROWS)
    def _(r):
        for k in range(D_H // LANES):
            histbuf[r, pl.ds(k * LANES, LANES)] = zeros16

    for k in range(H_ROWS // LANES):
        iotabuf[pl.ds(k * LANES, LANES)] = lane16 + (k * LANES)

    @pl.loop(sid, ZB, step=NS)
    def _(b):
        pltpu.sync_copy(pbA, s_shared.at[pl.ds(b * C, C)])

    @pl.when(sid == 1)
    def _():
        pltpu.sync_copy(histbuf, cnt_shared)

    plsc.subcore_barrier()

    cbs = (cb0, cb1, cb2, cb3)
    rbs = (rb0, rb1, rb2, rb3)
    css = (cs0, cs1, cs2, cs3)

    def fetch_colrow(chunk, q):
        base = chunk * C
        pltpu.async_copy(col_hbm.at[pl.ds(base, C)], cbs[q], css[q])
        pltpu.async_copy(row_hbm.at[pl.ds(base, C)], rbs[q], css[q])

    def do_chunk(j, t):
        chunk = wid + NW * j
        pb, eb, ds, ss = ((pbA, ebA, dsA, ssA) if t % 2 == 0
                          else (pbB, ebB, dsB, ssB))
        npb, neb, nds, nss = ((pbA, ebA, dsA, ssA) if t % 2 == 1
                             else (pbB, ebB, dsB, ssB))
        q, nq, q2 = t % 4, (t + 1) % 4, (t + 2) % 4

        pltpu.make_async_copy(p_hbm.at[cbs[q]], pb, ds).wait()
        pltpu.make_async_copy(e_hbm.at[pl.ds(0, C)], eb, ds).wait()

        @pl.when(j >= 1)
        def _():
            pltpu.make_async_copy(npb, s_shared.at[rbs[q]], nss).wait()

        @pl.when(j + 1 < NPT)
        def _():
            pltpu.make_async_copy(col_hbm.at[pl.ds(0, C)], cbs[nq],
                                  css[nq]).wait()
            pltpu.make_async_copy(row_hbm.at[pl.ds(0, C)], rbs[nq],
                                  css[nq]).wait()
            pltpu.async_copy(p_hbm.at[cbs[nq]], npb, nds)
            pltpu.async_copy(e_hbm.at[pl.ds((chunk + NW) * C, C)], neb, nds)

        @pl.when(j + 2 < NPT)
        def _():
            fetch_colrow(chunk + 2 * NW, q2)

        @plsc.parallel_loop(0, C, unroll=2)
        def _(r):
            for k in range(D_H // LANES):
                sl = pl.ds(k * LANES, LANES)
                pb[r, sl] = jnp.maximum(pb[r, sl] + eb[r, sl], 0.0)

        for k in range(C // LANES):
            rv = rbs[q][pl.ds(k * LANES, LANES)]
            plsc.addupdate_scatter(
                histbuf, [lax.shift_right_logical(rv, 7),
                          lax.bitwise_and(rv, 127)], ones16)
        if C % LANES:
            rv = rbs[q][pl.ds(C - LANES, LANES)]
            plsc.addupdate_scatter(
                histbuf, [lax.shift_right_logical(rv, 7),
                          lax.bitwise_and(rv, 127)], ones16,
                mask=lane16 >= (LANES - C % LANES))

        pltpu.async_copy(pb, s_shared.at[rbs[q]], ss, add=True)

    pltpu.sync_copy(col_hbm.at[pl.ds(wid * C, C)], cb0)
    pltpu.sync_copy(row_hbm.at[pl.ds(wid * C, C)], rb0)
    pltpu.async_copy(p_hbm.at[cb0], pbA, dsA)
    pltpu.async_copy(e_hbm.at[pl.ds(wid * C, C)], ebA, dsA)
    fetch_colrow(wid + NW, 1)

    @pl.loop(0, NPT - 2, step=4)
    def _(jj):
        do_chunk(jj, 0)
        do_chunk(jj + 1, 1)
        do_chunk(jj + 2, 2)
        do_chunk(jj + 3, 3)

    do_chunk(NPT - 2, 0)
    do_chunk(NPT - 1, 1)

    pltpu.make_async_copy(pbB, s_shared.at[rb1], ssB).wait()

    pltpu.sync_copy(histbuf, cnt_shared.at[iotabuf], add=True)

    plsc.subcore_barrier()

    @pl.loop(sid, ZB, step=NS)
    def _(b):
        pltpu.sync_copy(s_shared.at[pl.ds(b * C, C)],
                        s_out.at[cid].at[pl.ds(b * C, C)])

    @pl.when(sid == 1)
    def _():
        pltpu.sync_copy(cnt_shared, cnt_out.at[cid])


def _node_proj_body(nf_ref, w_ref, b_ref, out_ref):
    out_ref[...] = lax.dot_general(
        nf_ref[...], w_ref[...], (((1,), (0,)), ((), ())),
        preferred_element_type=jnp.float32, precision=HIGH) + b_ref[...]


def _edge_proj_body(ea_ref, w_ref, out_ref):
    out_ref[...] = lax.dot_general(
        ea_ref[...], w_ref[...], (((1,), (0,)), ((), ())),
        preferred_element_type=jnp.float32,
        precision=jax.lax.Precision.DEFAULT)


def _final_body(p0_ref, p1_ref, c0_ref, c1_ref, nf_ref, w2_ref, b2_ref,
                w3n_ref, w3m_ref, b3_ref, w4_ref, b4_ref, out_ref):
    s = p0_ref[...] + p1_ref[...]
    cnt = c0_ref[...] + c1_ref[...]
    sum_t = lax.dot_general(s, w2_ref[...], (((1,), (0,)), ((), ())),
                            preferred_element_type=jnp.float32,
                            precision=HIGH) + cnt * b2_ref[...]
    agg = sum_t / jnp.maximum(cnt, 1.0)
    u = lax.dot_general(nf_ref[...], w3n_ref[...], (((1,), (0,)), ((), ())),
                        preferred_element_type=jnp.float32, precision=HIGH)
    u = u + lax.dot_general(agg, w3m_ref[...], (((1,), (0,)), ((), ())),
                            preferred_element_type=jnp.float32,
                            precision=HIGH) + b3_ref[...]
    u = jnp.maximum(u, 0.0)
    out_ref[...] = lax.dot_general(
        u, w4_ref[...], (((1,), (0,)), ((), ())),
        preferred_element_type=jnp.float32, precision=HIGH) + b4_ref[...]


def kernel(node_feat, edge_index, edge_attr, W1, b1, W2, b2, W3, b3, W4, b4):
    row = edge_index[0]
    col = edge_index[1]
    w1n, w1e = W1[:D_IN], W1[D_IN:]
    w3n, w3m = W3[:D_IN], W3[D_IN:]

    p = pl.pallas_call(
        _node_proj_body,
        out_shape=jax.ShapeDtypeStruct((N_NODES, D_H), jnp.float32),
    )(node_feat, w1n, b1.reshape(1, D_H))

    eb = 4000
    e = pl.pallas_call(
        _edge_proj_body,
        grid=(N_EDGES // eb,),
        in_specs=[pl.BlockSpec((eb, 16), lambda i: (i, 0)),
                  pl.BlockSpec((16, D_H), lambda i: (0, 0))],
        out_specs=pl.BlockSpec((eb, D_H), lambda i: (i, 0)),
        out_shape=jax.ShapeDtypeStruct((N_EDGES, D_H), jnp.float32),
    )(edge_attr, w1e)

    partials, counts = _sc_segment_kernel(p, e, col, row)
    cnt0 = counts[0].reshape(H_ROWS * D_H, 1)[:N_NODES]
    cnt1 = counts[1].reshape(H_ROWS * D_H, 1)[:N_NODES]

    nb = 1000
    out = pl.pallas_call(
        _final_body,
        grid=(N_NODES // nb,),
        in_specs=[pl.BlockSpec((nb, D_H), lambda i: (i, 0)),
                  pl.BlockSpec((nb, D_H), lambda i: (i, 0)),
                  pl.BlockSpec((nb, 1), lambda i: (i, 0)),
                  pl.BlockSpec((nb, 1), lambda i: (i, 0)),
                  pl.BlockSpec((nb, D_IN), lambda i: (i, 0)),
                  pl.BlockSpec((D_H, D_H), lambda i: (0, 0)),
                  pl.BlockSpec((1, D_H), lambda i: (0, 0)),
                  pl.BlockSpec((D_IN, D_H), lambda i: (0, 0)),
                  pl.BlockSpec((D_H, D_H), lambda i: (0, 0)),
                  pl.BlockSpec((1, D_H), lambda i: (0, 0)),
                  pl.BlockSpec((D_H, D_H), lambda i: (0, 0)),
                  pl.BlockSpec((1, D_H), lambda i: (0, 0))],
        out_specs=pl.BlockSpec((nb, D_H), lambda i: (i, 0)),
        out_shape=jax.ShapeDtypeStruct((N_NODES, D_H), jnp.float32),
    )(partials[0], partials[1], cnt0, cnt1, node_feat, W2, b2.reshape(1, D_H),
      w3n, w3m, b3.reshape(1, D_H), W4, b4.reshape(1, D_H))

    return out


# speedup vs baseline: 1.4342x; 1.0179x over previous
import dataclasses
import functools

import jax
import jax.numpy as jnp
from jax import lax
from jax.experimental import pallas as pl
from jax.experimental.pallas import tpu as pltpu
from jax.experimental.pallas import tpu_sc as plsc

N_NODES = 10000
N_EDGES = 320000
D_IN = 128
D_H = 128
NC = 2
NS = 16
NW = NC * NS
LANES = 16
C = 40
NPT = N_EDGES // (NW * C)
S_ROWS = 10240
ZB = S_ROWS // C
H_ROWS = 80
HIGH = jax.lax.Precision.HIGHEST

_sc_mesh = plsc.VectorSubcoreMesh(
    core_axis_name="c", subcore_axis_name="s", num_cores=NC, num_subcores=NS)

_sc_params = pltpu.CompilerParams()
_flds = pltpu.CompilerParams.__dataclass_fields__
if "needs_layout_passes" in _flds:
    _sc_params = dataclasses.replace(_sc_params, needs_layout_passes=False)


@functools.partial(
    pl.kernel,
    out_type=(jax.ShapeDtypeStruct((NC, S_ROWS, D_H), jnp.float32),
              jax.ShapeDtypeStruct((NC, H_ROWS, D_H), jnp.float32)),
    mesh=_sc_mesh,
    compiler_params=_sc_params,
    scratch_types=[
        pltpu.VMEM((C,), jnp.int32), pltpu.VMEM((C,), jnp.int32),
        pltpu.VMEM((C,), jnp.int32), pltpu.VMEM((C,), jnp.int32),
        pltpu.VMEM((C,), jnp.int32), pltpu.VMEM((C,), jnp.int32),
        pltpu.VMEM((C,), jnp.int32), pltpu.VMEM((C,), jnp.int32),
        pltpu.VMEM((C, D_H), jnp.float32), pltpu.VMEM((C, D_H), jnp.float32),
        pltpu.VMEM((C, D_H), jnp.float32), pltpu.VMEM((C, D_H), jnp.float32),
        pltpu.VMEM((H_ROWS, D_H), jnp.float32),
        pltpu.VMEM((H_ROWS,), jnp.int32),
        pltpu.SemaphoreType.DMA, pltpu.SemaphoreType.DMA,
        pltpu.SemaphoreType.DMA, pltpu.SemaphoreType.DMA,
        pltpu.SemaphoreType.DMA, pltpu.SemaphoreType.DMA,
        pltpu.SemaphoreType.DMA, pltpu.SemaphoreType.DMA,
        pltpu.VMEM_SHARED((S_ROWS, D_H), jnp.float32),
        pltpu.VMEM_SHARED((H_ROWS, D_H), jnp.float32),
    ],
)
def _sc_segment_kernel(p_hbm, e_hbm, col_hbm, row_hbm, s_out, cnt_out,
                       cb0, cb1, cb2, cb3, rb0, rb1, rb2, rb3,
                       pbA, pbB, ebA, ebB, histbuf, iotabuf,
                       dsA, dsB, cs0, cs1, cs2, cs3, ssA, ssB,
                       s_shared, cnt_shared):
    cid = lax.axis_index("c")
    sid = lax.axis_index("s")
    wid = sid * NC + cid
    zeros16 = jnp.zeros((LANES,), jnp.float32)
    ones16 = jnp.ones((LANES,), jnp.float32)
    lane16 = lax.iota(jnp.int32, 16)

    @pl.loop(0, C)
    def _(r):
        for k in range(D_H // LANES):
            pbA[r, pl.ds(k * LANES, LANES)] = zeros16

    @pl.loop(0, H_ROWS)
    def _(r):
        for k in range(D_H // LANES):
            histbuf[r, pl.ds(k * LANES, LANES)] = zeros16

    for k in range(H_ROWS // LANES):
        iotabuf[pl.ds(k * LANES, LANES)] = lane16 + (k * LANES)

    @pl.loop(sid, ZB, step=NS)
    def _(b):
        pltpu.sync_copy(pbA, s_shared.at[pl.ds(b * C, C)])

    @pl.when(sid == 1)
    def _():
        pltpu.sync_copy(histbuf, cnt_shared)

    plsc.subcore_barrier()

    cbs = (cb0, cb1, cb2, cb3)
    rbs = (rb0, rb1, rb2, rb3)
    css = (cs0, cs1, cs2, cs3)

    def fetch_colrow(chunk, q):
        base = chunk * C
        pltpu.async_copy(col_hbm.at[pl.ds(base, C)], cbs[q], css[q])
        pltpu.async_copy(row_hbm.at[pl.ds(base, C)], rbs[q], css[q])

    def do_chunk(j, t):
        chunk = wid + NW * j
        pb, eb, ds, ss = ((pbA, ebA, dsA, ssA) if t % 2 == 0
                          else (pbB, ebB, dsB, ssB))
        npb, neb, nds, nss = ((pbA, ebA, dsA, ssA) if t % 2 == 1
                             else (pbB, ebB, dsB, ssB))
        q, nq, q2 = t % 4, (t + 1) % 4, (t + 2) % 4

        pltpu.make_async_copy(p_hbm.at[cbs[q]], pb, ds).wait()
        pltpu.make_async_copy(e_hbm.at[pl.ds(0, C)], eb, ds).wait()

        @pl.when(j >= 1)
        def _():
            pltpu.make_async_copy(npb, s_shared.at[rbs[q]], nss).wait()

        @pl.when(j + 1 < NPT)
        def _():
            pltpu.make_async_copy(col_hbm.at[pl.ds(0, C)], cbs[nq],
                                  css[nq]).wait()
            pltpu.make_async_copy(row_hbm.at[pl.ds(0, C)], rbs[nq],
                                  css[nq]).wait()
            pltpu.async_copy(p_hbm.at[cbs[nq]], npb, nds)
            pltpu.async_copy(e_hbm.at[pl.ds((chunk + NW) * C, C)], neb, nds)

        @pl.when(j + 2 < NPT)
        def _():
            fetch_colrow(chunk + 2 * NW, q2)

        @plsc.parallel_loop(0, C, unroll=2)
        def _(r):
            for k in range(D_H // LANES):
                sl = pl.ds(k * LANES, LANES)
                pb[r, sl] = jnp.maximum(pb[r, sl] + eb[r, sl], 0.0)

        for k in range(C // LANES):
            rv = rbs[q][pl.ds(k * LANES, LANES)]
            plsc.addupdate_scatter(
                histbuf, [lax.shift_right_logical(rv, 7),
                          lax.bitwise_and(rv, 127)], ones16)
        if C % LANES:
            rv = rbs[q][pl.ds(C - LANES, LANES)]
            plsc.addupdate_scatter(
                histbuf, [lax.shift_right_logical(rv, 7),
                          lax.bitwise_and(rv, 127)], ones16,
                mask=lane16 >= (LANES - C % LANES))

        pltpu.async_copy(pb, s_shared.at[rbs[q]], ss, add=True)

    pltpu.sync_copy(col_hbm.at[pl.ds(wid * C, C)], cb0)
    pltpu.sync_copy(row_hbm.at[pl.ds(wid * C, C)], rb0)
    pltpu.async_copy(p_hbm.at[cb0], pbA, dsA)
    pltpu.async_copy(e_hbm.at[pl.ds(wid * C, C)], ebA, dsA)
    fetch_colrow(wid + NW, 1)

    @pl.loop(0, NPT - 2, step=4)
    def _(jj):
        do_chunk(jj, 0)
        do_chunk(jj + 1, 1)
        do_chunk(jj + 2, 2)
        do_chunk(jj + 3, 3)

    do_chunk(NPT - 2, 0)
    do_chunk(NPT - 1, 1)

    pltpu.make_async_copy(pbB, s_shared.at[rb1], ssB).wait()

    pltpu.sync_copy(histbuf, cnt_shared.at[iotabuf], add=True)

    plsc.subcore_barrier()

    @pl.loop(sid, ZB, step=NS)
    def _(b):
        pltpu.sync_copy(s_shared.at[pl.ds(b * C, C)],
                        s_out.at[cid].at[pl.ds(b * C, C)])

    @pl.when(sid == 1)
    def _():
        pltpu.sync_copy(cnt_shared, cnt_out.at[cid])


def _node_proj_body(nf_ref, w_ref, b_ref, out_ref):
    out_ref[...] = lax.dot_general(
        nf_ref[...], w_ref[...], (((1,), (0,)), ((), ())),
        preferred_element_type=jnp.float32, precision=HIGH) + b_ref[...]


def _edge_proj_body(ea_ref, w_ref, out_ref):
    out_ref[...] = lax.dot_general(
        ea_ref[...].astype(jnp.bfloat16), w_ref[...].astype(jnp.bfloat16),
        (((1,), (0,)), ((), ())), preferred_element_type=jnp.float32)


def _final_body(p0_ref, p1_ref, c0_ref, c1_ref, nf_ref, w2_ref, b2_ref,
                w3n_ref, w3m_ref, b3_ref, w4_ref, b4_ref, out_ref):
    s = p0_ref[...] + p1_ref[...]
    h = c0_ref[...] + c1_ref[...]
    nb, hb = s.shape[0], h.shape[0]
    r_id = lax.broadcasted_iota(jnp.int32, (nb, hb), 0) // D_H
    onehot_r = (r_id == lax.broadcasted_iota(jnp.int32, (nb, hb), 1))
    m = lax.dot_general(onehot_r.astype(jnp.float32), h,
                        (((1,), (0,)), ((), ())),
                        preferred_element_type=jnp.float32, precision=HIGH)
    lane = lax.broadcasted_iota(jnp.int32, (nb, D_H), 1)
    nmod = lax.broadcasted_iota(jnp.int32, (nb, D_H), 0) % D_H
    cnt = jnp.sum(jnp.where(lane == nmod, m, 0.0), axis=1, keepdims=True)
    sum_t = lax.dot_general(s, w2_ref[...], (((1,), (0,)), ((), ())),
                            preferred_element_type=jnp.float32,
                            precision=HIGH) + cnt * b2_ref[...]
    agg = sum_t / jnp.maximum(cnt, 1.0)
    u = lax.dot_general(nf_ref[...], w3n_ref[...], (((1,), (0,)), ((), ())),
                        preferred_element_type=jnp.float32, precision=HIGH)
    u = u + lax.dot_general(agg, w3m_ref[...], (((1,), (0,)), ((), ())),
                            preferred_element_type=jnp.float32,
                            precision=HIGH) + b3_ref[...]
    u = jnp.maximum(u, 0.0)
    out_ref[...] = lax.dot_general(
        u, w4_ref[...], (((1,), (0,)), ((), ())),
        preferred_element_type=jnp.float32, precision=HIGH) + b4_ref[...]


def kernel(node_feat, edge_index, edge_attr, W1, b1, W2, b2, W3, b3, W4, b4):
    row = edge_index[0]
    col = edge_index[1]
    w1n, w1e = W1[:D_IN], W1[D_IN:]
    w3n, w3m = W3[:D_IN], W3[D_IN:]

    p = pl.pallas_call(
        _node_proj_body,
        out_shape=jax.ShapeDtypeStruct((N_NODES, D_H), jnp.float32),
    )(node_feat, w1n, b1.reshape(1, D_H))

    eb = 4000
    e = pl.pallas_call(
        _edge_proj_body,
        grid=(N_EDGES // eb,),
        in_specs=[pl.BlockSpec((eb, 16), lambda i: (i, 0)),
                  pl.BlockSpec((16, D_H), lambda i: (0, 0))],
        out_specs=pl.BlockSpec((eb, D_H), lambda i: (i, 0)),
        out_shape=jax.ShapeDtypeStruct((N_EDGES, D_H), jnp.float32),
    )(edge_attr, w1e)

    partials, counts = _sc_segment_kernel(p, e, col, row)
    nf_pad = jnp.concatenate(
        [node_feat, jnp.zeros((S_ROWS - N_NODES, D_IN), jnp.float32)])

    nb = 1024
    hb = nb // D_H
    out = pl.pallas_call(
        _final_body,
        grid=(S_ROWS // nb,),
        in_specs=[pl.BlockSpec((nb, D_H), lambda i: (i, 0)),
                  pl.BlockSpec((nb, D_H), lambda i: (i, 0)),
                  pl.BlockSpec((hb, D_H), lambda i: (i, 0)),
                  pl.BlockSpec((hb, D_H), lambda i: (i, 0)),
                  pl.BlockSpec((nb, D_IN), lambda i: (i, 0)),
                  pl.BlockSpec((D_H, D_H), lambda i: (0, 0)),
                  pl.BlockSpec((1, D_H), lambda i: (0, 0)),
                  pl.BlockSpec((D_IN, D_H), lambda i: (0, 0)),
                  pl.BlockSpec((D_H, D_H), lambda i: (0, 0)),
                  pl.BlockSpec((1, D_H), lambda i: (0, 0)),
                  pl.BlockSpec((D_H, D_H), lambda i: (0, 0)),
                  pl.BlockSpec((1, D_H), lambda i: (0, 0))],
        out_specs=pl.BlockSpec((nb, D_H), lambda i: (i, 0)),
        out_shape=jax.ShapeDtypeStruct((S_ROWS, D_H), jnp.float32),
    )(partials[0], partials[1], counts[0], counts[1], nf_pad, W2,
      b2.reshape(1, D_H), w3n, w3m, b3.reshape(1, D_H), W4,
      b4.reshape(1, D_H))

    return out[:N_NODES]
